# baseline (speedup 1.0000x reference)
import functools

import jax
import jax.numpy as jnp
from jax import lax
from jax.experimental import pallas as pl
from jax.experimental.pallas import tpu as pltpu
from jax.experimental.pallas import tpu_sc as plsc

N = 10000
E = 320000
IN = 128
H = 128
ED = 16
K = 32

NC = 2
NS = 16
NW = NC * NS
C = 64
CPW = 159
EPW = C * CPW
E_PAD = NW * EPW
N_PAD = 10112
ROWS_PER_TILE = N_PAD // NS


def _sc_msg_pass(h_hbm, e_hbm, src_hbm, dst_hbm, out_hbm,
                 sidx0, sidx1, sidx2, didx0, didx1, didx2,
                 ebuf0, ebuf1, ebuf2, hbuf0, hbuf1, hbuf2,
                 aggr,
                 se0, se1, se2, sh0, sh1, sh2,
                 si0, si1, si2):
    cid = lax.axis_index("c")
    sid = lax.axis_index("s")
    wid = sid * NC + cid
    base0 = wid * EPW

    sidx = [sidx0, sidx1, sidx2]
    didx = [didx0, didx1, didx2]
    ebuf = [ebuf0, ebuf1, ebuf2]
    hbuf = [hbuf0, hbuf1, hbuf2]
    sem_e = [se0, se1, se2]
    sem_h = [sh0, sh1, sh2]
    sem_i = [si0, si1, si2]

    zeros16 = jnp.zeros((16,), jnp.float32)

    @plsc.parallel_loop(0, C, unroll=4)
    def _(j):
        for g in range(H // 16):
            ebuf0[j, pl.ds(g * 16, 16)] = zeros16

    row0 = sid * ROWS_PER_TILE
    full = ROWS_PER_TILE // C
    rem = ROWS_PER_TILE - full * C
    for r in range(full):
        pltpu.sync_copy(ebuf0, aggr.at[pl.ds(row0 + r * C, C)])
    pltpu.sync_copy(ebuf0.at[pl.ds(0, rem)], aggr.at[pl.ds(row0 + full * C, rem)])
    plsc.subcore_barrier()

    def start_idx(chunk, b):
        pltpu.async_copy(src_hbm.at[pl.ds(base0 + chunk * C, C)], sidx[b], sem_i[b])
        pltpu.async_copy(dst_hbm.at[pl.ds(base0 + chunk * C, C)], didx[b], sem_i[b])

    def wait_idx(b):
        pltpu.make_async_copy(src_hbm.at[pl.ds(0, C)], sidx[b], sem_i[b]).wait()
        pltpu.make_async_copy(dst_hbm.at[pl.ds(0, C)], didx[b], sem_i[b]).wait()

    def start_streams(chunk, b):
        pltpu.async_copy(e_hbm.at[pl.ds(base0 + chunk * C, C)], ebuf[b], sem_e[b])
        pltpu.async_copy(h_hbm.at[sidx[b]], hbuf[b], sem_h[b])

    def wait_streams(b):
        pltpu.make_async_copy(e_hbm.at[pl.ds(0, C)], ebuf[b], sem_e[b]).wait()
        pltpu.make_async_copy(h_hbm.at[sidx[b]], hbuf[b], sem_h[b]).wait()

    start_idx(0, 0)
    wait_idx(0)
    start_streams(0, 0)

    NG = CPW // 3

    def pipe_body(g, _):
        for k in range(3):
            i = 3 * g + k
            cur = k
            nxt = (k + 1) % 3

            def launch_streams(i=i, nxt=nxt):
                wait_idx(nxt)
                start_streams(i + 1, nxt)

            last = k == 2
            if not last:
                start_idx(i + 1, nxt)
            else:
                @pl.when(g < NG - 1)
                def _():
                    start_idx(i + 1, nxt)

            wait_streams(cur)

            @plsc.parallel_loop(0, C, unroll=4)
            def _(j, cur=cur):
                for gg in range(H // 16):
                    sl = pl.ds(gg * 16, 16)
                    ebuf[cur][j, sl] = jnp.maximum(
                        ebuf[cur][j, sl] + hbuf[cur][j, sl], 0.0)

            if not last:
                launch_streams()
            else:
                @pl.when(g < NG - 1)
                def _():
                    launch_streams()

            pltpu.sync_copy(ebuf[cur], aggr.at[didx[cur]], add=True)
        return 0

    lax.fori_loop(0, NG, pipe_body, 0)
    plsc.subcore_barrier()

    out_row0 = cid * N_PAD + row0
    for r in range(full):
        pltpu.sync_copy(aggr.at[pl.ds(row0 + r * C, C)], hbuf0)
        pltpu.sync_copy(hbuf0, out_hbm.at[pl.ds(out_row0 + r * C, C)])
    pltpu.sync_copy(aggr.at[pl.ds(row0 + full * C, rem)], hbuf0.at[pl.ds(0, rem)])
    pltpu.sync_copy(hbuf0.at[pl.ds(0, rem)], out_hbm.at[pl.ds(out_row0 + full * C, rem)])


_sc_mesh = plsc.VectorSubcoreMesh(core_axis_name="c", subcore_axis_name="s",
                                  num_cores=NC, num_subcores=NS)

_sc_msg_pass_call = functools.partial(
    pl.kernel,
    out_type=jax.ShapeDtypeStruct((NC * N_PAD, H), jnp.float32),
    mesh=_sc_mesh,
    scratch_types=(
        [pltpu.VMEM((C,), jnp.int32)] * 6
        + [pltpu.VMEM((C, H), jnp.float32)] * 6
        + [pltpu.VMEM_SHARED((N_PAD, H), jnp.float32)]
        + [pltpu.SemaphoreType.DMA] * 9
    ),
)(_sc_msg_pass)


def _edgelin_body(ea_ref, w1t_ref, b1_ref, w2t_ref, b2_ref, e1_ref, e2_ref):
    a = ea_ref[...]
    e1_ref[...] = jnp.dot(a, w1t_ref[...], preferred_element_type=jnp.float32) + b1_ref[...]
    e2_ref[...] = jnp.dot(a, w2t_ref[...], preferred_element_type=jnp.float32) + b2_ref[...]


_BE = 1024


def _edgelin(ea_pad, w1t, b1, w2t, b2):
    grid = (E_PAD // _BE,)
    return pl.pallas_call(
        _edgelin_body,
        grid=grid,
        in_specs=[
            pl.BlockSpec((_BE, ED), lambda i: (i, 0)),
            pl.BlockSpec((ED, H), lambda i: (0, 0)),
            pl.BlockSpec((1, H), lambda i: (0, 0)),
            pl.BlockSpec((ED, H), lambda i: (0, 0)),
            pl.BlockSpec((1, H), lambda i: (0, 0)),
        ],
        out_specs=[
            pl.BlockSpec((_BE, H), lambda i: (i, 0)),
            pl.BlockSpec((_BE, H), lambda i: (i, 0)),
        ],
        out_shape=[
            jax.ShapeDtypeStruct((E_PAD, H), jnp.float32),
            jax.ShapeDtypeStruct((E_PAD, H), jnp.float32),
        ],
    )(ea_pad, w1t, b1, w2t, b2)


def _mlp1_body(x_ref, a0_ref, a1_ref, w1t_ref, b1_ref, w2t_ref, b2_ref, out_ref):
    z = x_ref[...] + a0_ref[...] + a1_ref[...]
    h = jax.nn.relu(jnp.dot(z, w1t_ref[...], preferred_element_type=jnp.float32) + b1_ref[...])
    o = jnp.dot(h, w2t_ref[...], preferred_element_type=jnp.float32) + b2_ref[...]
    out_ref[...] = jax.nn.relu(o)


def _mlp1(x, a0, a1, w1t, b1, w2t, b2):
    return pl.pallas_call(
        _mlp1_body,
        out_shape=jax.ShapeDtypeStruct((N, H), jnp.float32),
    )(x, a0, a1, w1t, b1, w2t, b2)


def _head_body(idx_ref, h1_ref, a0_ref, a1_ref,
               mw1t_ref, mb1_ref, mw2t_ref, mb2_ref,
               wct_ref, wdt_ref, wnt_ref, hb1_ref, hw2_ref, hb2_ref,
               out_ref, rows_ref):
    for k in [0, 1] + list(range(8, 40)):
        idx = idx_ref[k]
        r = (h1_ref[pl.ds(idx, 1), :] + a0_ref[pl.ds(idx, 1), :]
             + a1_ref[pl.ds(idx, 1), :])
        rows_ref[pl.ds(k, 1), :] = r
    rows = rows_ref[...]
    hmid = jax.nn.relu(jnp.dot(rows, mw1t_ref[...], preferred_element_type=jnp.float32) + mb1_ref[...])
    h2r = jnp.dot(hmid, mw2t_ref[...], preferred_element_type=jnp.float32) + mb2_ref[...]
    curr = h2r[0:1, :]
    dest = h2r[1:2, :]
    nbr = h2r[8:40, :]
    base = (jnp.dot(curr, wct_ref[...], preferred_element_type=jnp.float32)
            + jnp.dot(dest, wdt_ref[...], preferred_element_type=jnp.float32)
            + hb1_ref[...])
    hh = jax.nn.relu(jnp.dot(nbr, wnt_ref[...], preferred_element_type=jnp.float32) + base)
    q = jnp.sum(hh * hw2_ref[...], axis=1, keepdims=True) + hb2_ref[0, 0]
    out_ref[...] = q


def _head(idx40, h1, a0, a1, mw1t, mb1, mw2t, mb2,
          wct, wdt, wnt, hb1, hw2, hb2):
    return pl.pallas_call(
        _head_body,
        in_specs=[
            pl.BlockSpec(memory_space=pltpu.SMEM),
            pl.BlockSpec((N, H), lambda: (0, 0)),
            pl.BlockSpec((N, H), lambda: (0, 0)),
            pl.BlockSpec((N, H), lambda: (0, 0)),
            pl.BlockSpec((H, H), lambda: (0, 0)),
            pl.BlockSpec((1, H), lambda: (0, 0)),
            pl.BlockSpec((H, H), lambda: (0, 0)),
            pl.BlockSpec((1, H), lambda: (0, 0)),
            pl.BlockSpec((H, H), lambda: (0, 0)),
            pl.BlockSpec((H, H), lambda: (0, 0)),
            pl.BlockSpec((H, H), lambda: (0, 0)),
            pl.BlockSpec((1, H), lambda: (0, 0)),
            pl.BlockSpec((1, H), lambda: (0, 0)),
            pl.BlockSpec((1, 1), lambda: (0, 0)),
        ],
        out_shape=jax.ShapeDtypeStruct((K, 1), jnp.float32),
        scratch_shapes=[pltpu.VMEM((40, H), jnp.float32)],
    )(idx40, h1, a0, a1, mw1t, mb1, mw2t, mb2, wct, wdt, wnt, hb1, hw2, hb2)


def kernel(x, edge_index, curr_idx, dest_idx, neighbor_indices, edge_attr,
           lin_e1_W, lin_e1_b, mlp1_W1, mlp1_b1, mlp1_W2, mlp1_b2,
           lin_e2_W, lin_e2_b, mlp2_W1, mlp2_b1, mlp2_W2, mlp2_b2,
           head_W1, head_b1, head_W2, head_b2):
    src = edge_index[0]
    dst = edge_index[1]
    pad = E_PAD - E
    src_pad = jnp.concatenate([src, jnp.zeros((pad,), jnp.int32)])
    dst_pad = jnp.concatenate([dst, jnp.full((pad,), N, jnp.int32)])
    ea_pad = jnp.concatenate([edge_attr, jnp.zeros((pad, ED), jnp.float32)])

    e1, e2 = _edgelin(ea_pad, lin_e1_W.T, lin_e1_b[None, :],
                      lin_e2_W.T, lin_e2_b[None, :])

    a1_parts = _sc_msg_pass_call(x, e1, src_pad, dst_pad)
    a10 = a1_parts[:N]
    a11 = a1_parts[N_PAD:N_PAD + N]

    h1 = _mlp1(x, a10, a11, mlp1_W1.T, mlp1_b1[None, :],
               mlp1_W2.T, mlp1_b2[None, :])

    a2_parts = _sc_msg_pass_call(h1, e2, src_pad, dst_pad)
    a20 = a2_parts[:N]
    a21 = a2_parts[N_PAD:N_PAD + N]

    ci = jnp.asarray(curr_idx, jnp.int32)[None]
    di = jnp.asarray(dest_idx, jnp.int32)[None]
    idx40 = jnp.concatenate([ci, di, jnp.zeros((6,), jnp.int32),
                             neighbor_indices.astype(jnp.int32)])

    wct = head_W1[:, 0:H].T
    wdt = head_W1[:, H:2 * H].T
    wnt = head_W1[:, 2 * H:3 * H].T

    q = _head(idx40, h1, a20, a21,
              mlp2_W1.T, mlp2_b1[None, :], mlp2_W2.T, mlp2_b2[None, :],
              wct, wdt, wnt, head_b1[None, :], head_W2, head_b2[None, :])
    return q[:, 0]


# speedup vs baseline: 1.3938x; 1.3938x over previous
import functools

import jax
import jax.numpy as jnp
from jax import lax
from jax.experimental import pallas as pl
from jax.experimental.pallas import tpu as pltpu
from jax.experimental.pallas import tpu_sc as plsc

N = 10000
E = 320000
IN = 128
H = 128
ED = 16
K = 32

NC = 2
NS = 16
NW = NC * NS
C = 64
CPW = 159
EPW = C * CPW
E_PAD = NW * EPW
N_PAD = 10112
ROWS_PER_TILE = N_PAD // NS
REP = 8


def _sc_msg_pass(h_hbm, e_hbm, src_hbm, dst_hbm, out_hbm,
                 sidx0, sidx1, sidx2, didx0, didx1, didx2,
                 ebuf0, ebuf1, ebuf2, hbuf0, hbuf1, hbuf2,
                 aggr,
                 se0, se1, se2, sh0, sh1, sh2,
                 si0, si1, si2):
    cid = lax.axis_index("c")
    sid = lax.axis_index("s")
    wid = sid * NC + cid
    base0 = wid * EPW

    sidx = [sidx0, sidx1, sidx2]
    didx = [didx0, didx1, didx2]
    ebuf = [ebuf0, ebuf1, ebuf2]
    hbuf = [hbuf0, hbuf1, hbuf2]
    sem_e = [se0, se1, se2]
    sem_h = [sh0, sh1, sh2]
    sem_i = [si0, si1, si2]

    zeros16 = jnp.zeros((16,), jnp.float32)

    @plsc.parallel_loop(0, C, unroll=4)
    def _(j):
        for g in range(H // 16):
            ebuf0[j, pl.ds(g * 16, 16)] = zeros16

    row0 = sid * ROWS_PER_TILE
    full = ROWS_PER_TILE // C
    rem = ROWS_PER_TILE - full * C
    for r in range(full):
        pltpu.sync_copy(ebuf0, aggr.at[pl.ds(row0 + r * C, C)])
    pltpu.sync_copy(ebuf0.at[pl.ds(0, rem)], aggr.at[pl.ds(row0 + full * C, rem)])
    plsc.subcore_barrier()

    def start_idx(chunk, b):
        pltpu.async_copy(src_hbm.at[pl.ds(base0 + chunk * C, C)], sidx[b], sem_i[b])
        pltpu.async_copy(dst_hbm.at[pl.ds(base0 + chunk * C, C)], didx[b], sem_i[b])

    def wait_idx(b):
        pltpu.make_async_copy(src_hbm.at[pl.ds(0, C)], sidx[b], sem_i[b]).wait()
        pltpu.make_async_copy(dst_hbm.at[pl.ds(0, C)], didx[b], sem_i[b]).wait()

    def start_streams(chunk, b):
        pltpu.async_copy(e_hbm.at[pl.ds(base0 + chunk * C, C)], ebuf[b], sem_e[b])
        pltpu.async_copy(h_hbm.at[sidx[b]], hbuf[b], sem_h[b])

    def wait_streams(b):
        pltpu.make_async_copy(e_hbm.at[pl.ds(0, C)], ebuf[b], sem_e[b]).wait()
        pltpu.make_async_copy(h_hbm.at[sidx[b]], hbuf[b], sem_h[b]).wait()

    start_idx(0, 0)
    wait_idx(0)
    start_streams(0, 0)

    NG = CPW // 3

    def pipe_body(g, _):
        for k in range(3):
            i = 3 * g + k
            cur = k
            nxt = (k + 1) % 3

            def launch_streams(i=i, nxt=nxt):
                wait_idx(nxt)
                start_streams(i + 1, nxt)

            last = k == 2
            if not last:
                start_idx(i + 1, nxt)
            else:
                @pl.when(g < NG - 1)
                def _():
                    start_idx(i + 1, nxt)

            wait_streams(cur)

            @plsc.parallel_loop(0, C, unroll=4)
            def _(j, cur=cur):
                for gg in range(H // 16):
                    sl = pl.ds(gg * 16, 16)
                    ebuf[cur][j, sl] = jnp.maximum(
                        ebuf[cur][j, sl] + hbuf[cur][j, sl], 0.0)

            if not last:
                launch_streams()
            else:
                @pl.when(g < NG - 1)
                def _():
                    launch_streams()

            pltpu.sync_copy(ebuf[cur], aggr.at[didx[cur]], add=True)
        return 0

    lax.fori_loop(0, NG, pipe_body, 0)
    plsc.subcore_barrier()

    out_row0 = cid * N_PAD + row0
    for r in range(full):
        pltpu.sync_copy(aggr.at[pl.ds(row0 + r * C, C)], hbuf0)
        pltpu.sync_copy(hbuf0, out_hbm.at[pl.ds(out_row0 + r * C, C)])
    pltpu.sync_copy(aggr.at[pl.ds(row0 + full * C, rem)], hbuf0.at[pl.ds(0, rem)])
    pltpu.sync_copy(hbuf0.at[pl.ds(0, rem)], out_hbm.at[pl.ds(out_row0 + full * C, rem)])


_sc_mesh = plsc.VectorSubcoreMesh(core_axis_name="c", subcore_axis_name="s",
                                  num_cores=NC, num_subcores=NS)

_sc_msg_pass_call = functools.partial(
    pl.kernel,
    out_type=jax.ShapeDtypeStruct((NC * N_PAD, H), jnp.float32),
    mesh=_sc_mesh,
    scratch_types=(
        [pltpu.VMEM((C,), jnp.int32)] * 6
        + [pltpu.VMEM((C, H), jnp.float32)] * 6
        + [pltpu.VMEM_SHARED((N_PAD, H), jnp.float32)]
        + [pltpu.SemaphoreType.DMA] * 9
    ),
)(_sc_msg_pass)


def _edgelin_body(ea_ref, w1t_ref, b1_ref, w2t_ref, b2_ref, e1_ref, e2_ref):
    a = ea_ref[...]
    e1_ref[...] = jnp.dot(a, w1t_ref[...], preferred_element_type=jnp.float32) + b1_ref[...]
    e2_ref[...] = jnp.dot(a, w2t_ref[...], preferred_element_type=jnp.float32) + b2_ref[...]


_BE = 1024


def _edgelin(ea_pad, w1t, b1, w2t, b2):
    grid = (E_PAD // _BE,)
    return pl.pallas_call(
        _edgelin_body,
        grid=grid,
        in_specs=[
            pl.BlockSpec((_BE, ED), lambda i: (i, 0)),
            pl.BlockSpec((ED, H), lambda i: (0, 0)),
            pl.BlockSpec((1, H), lambda i: (0, 0)),
            pl.BlockSpec((ED, H), lambda i: (0, 0)),
            pl.BlockSpec((1, H), lambda i: (0, 0)),
        ],
        out_specs=[
            pl.BlockSpec((_BE, H), lambda i: (i, 0)),
            pl.BlockSpec((_BE, H), lambda i: (i, 0)),
        ],
        out_shape=[
            jax.ShapeDtypeStruct((E_PAD, H), jnp.float32),
            jax.ShapeDtypeStruct((E_PAD, H), jnp.float32),
        ],
    )(ea_pad, w1t, b1, w2t, b2)


def _replicate_body(x_ref, out_ref):
    out_ref[...] = x_ref[...]


def _replicate(x):
    return pl.pallas_call(
        _replicate_body,
        grid=(REP,),
        in_specs=[pl.BlockSpec((N, H), lambda r: (0, 0))],
        out_specs=pl.BlockSpec((N, H), lambda r: (r, 0)),
        out_shape=jax.ShapeDtypeStruct((REP * N, H), jnp.float32),
    )(x)


def _mlp1_body(x_ref, a0_ref, a1_ref, w1t_ref, b1_ref, w2t_ref, b2_ref, out_ref):
    z = x_ref[...] + a0_ref[...] + a1_ref[...]
    h = jax.nn.relu(jnp.dot(z, w1t_ref[...], preferred_element_type=jnp.float32) + b1_ref[...])
    o = jnp.dot(h, w2t_ref[...], preferred_element_type=jnp.float32) + b2_ref[...]
    out_ref[...] = jax.nn.relu(o)


def _mlp1(x, a0, a1, w1t, b1, w2t, b2):
    return pl.pallas_call(
        _mlp1_body,
        out_shape=jax.ShapeDtypeStruct((N, H), jnp.float32),
    )(x, a0, a1, w1t, b1, w2t, b2)


def _head_body(idx_ref, h1_ref, a0_ref, a1_ref,
               mw1t_ref, mb1_ref, mw2t_ref, mb2_ref,
               wct_ref, wdt_ref, wnt_ref, hb1_ref, hw2_ref, hb2_ref,
               out_ref, rows_ref):
    for k in [0, 1] + list(range(8, 40)):
        idx = idx_ref[k]
        r = (h1_ref[pl.ds(idx, 1), :] + a0_ref[pl.ds(idx, 1), :]
             + a1_ref[pl.ds(idx, 1), :])
        rows_ref[pl.ds(k, 1), :] = r
    rows = rows_ref[...]
    hmid = jax.nn.relu(jnp.dot(rows, mw1t_ref[...], preferred_element_type=jnp.float32) + mb1_ref[...])
    h2r = jnp.dot(hmid, mw2t_ref[...], preferred_element_type=jnp.float32) + mb2_ref[...]
    curr = h2r[0:1, :]
    dest = h2r[1:2, :]
    nbr = h2r[8:40, :]
    base = (jnp.dot(curr, wct_ref[...], preferred_element_type=jnp.float32)
            + jnp.dot(dest, wdt_ref[...], preferred_element_type=jnp.float32)
            + hb1_ref[...])
    hh = jax.nn.relu(jnp.dot(nbr, wnt_ref[...], preferred_element_type=jnp.float32) + base)
    q = jnp.sum(hh * hw2_ref[...], axis=1, keepdims=True) + hb2_ref[0, 0]
    out_ref[...] = q


def _head(idx40, h1, a0, a1, mw1t, mb1, mw2t, mb2,
          wct, wdt, wnt, hb1, hw2, hb2):
    return pl.pallas_call(
        _head_body,
        in_specs=[
            pl.BlockSpec(memory_space=pltpu.SMEM),
            pl.BlockSpec((N, H), lambda: (0, 0)),
            pl.BlockSpec((N, H), lambda: (0, 0)),
            pl.BlockSpec((N, H), lambda: (0, 0)),
            pl.BlockSpec((H, H), lambda: (0, 0)),
            pl.BlockSpec((1, H), lambda: (0, 0)),
            pl.BlockSpec((H, H), lambda: (0, 0)),
            pl.BlockSpec((1, H), lambda: (0, 0)),
            pl.BlockSpec((H, H), lambda: (0, 0)),
            pl.BlockSpec((H, H), lambda: (0, 0)),
            pl.BlockSpec((H, H), lambda: (0, 0)),
            pl.BlockSpec((1, H), lambda: (0, 0)),
            pl.BlockSpec((1, H), lambda: (0, 0)),
            pl.BlockSpec((1, 1), lambda: (0, 0)),
        ],
        out_shape=jax.ShapeDtypeStruct((K, 1), jnp.float32),
        scratch_shapes=[pltpu.VMEM((40, H), jnp.float32)],
    )(idx40, h1, a0, a1, mw1t, mb1, mw2t, mb2, wct, wdt, wnt, hb1, hw2, hb2)


def kernel(x, edge_index, curr_idx, dest_idx, neighbor_indices, edge_attr,
           lin_e1_W, lin_e1_b, mlp1_W1, mlp1_b1, mlp1_W2, mlp1_b2,
           lin_e2_W, lin_e2_b, mlp2_W1, mlp2_b1, mlp2_W2, mlp2_b2,
           head_W1, head_b1, head_W2, head_b2):
    src = edge_index[0]
    dst = edge_index[1]
    pad = E_PAD - E
    src_pad = jnp.concatenate([src, jnp.zeros((pad,), jnp.int32)])
    dst_pad = jnp.concatenate([dst, jnp.full((pad,), N, jnp.int32)])
    ea_pad = jnp.concatenate([edge_attr, jnp.zeros((pad, ED), jnp.float32)])
    src_rep = src_pad + (jnp.arange(E_PAD, dtype=jnp.int32) % REP) * N

    e1, e2 = _edgelin(ea_pad, lin_e1_W.T, lin_e1_b[None, :],
                      lin_e2_W.T, lin_e2_b[None, :])

    x_rep = _replicate(x)
    a1_parts = _sc_msg_pass_call(x_rep, e1, src_rep, dst_pad)
    a10 = a1_parts[:N]
    a11 = a1_parts[N_PAD:N_PAD + N]

    h1 = _mlp1(x, a10, a11, mlp1_W1.T, mlp1_b1[None, :],
               mlp1_W2.T, mlp1_b2[None, :])

    h1_rep = _replicate(h1)
    a2_parts = _sc_msg_pass_call(h1_rep, e2, src_rep, dst_pad)
    a20 = a2_parts[:N]
    a21 = a2_parts[N_PAD:N_PAD + N]

    ci = jnp.asarray(curr_idx, jnp.int32)[None]
    di = jnp.asarray(dest_idx, jnp.int32)[None]
    idx40 = jnp.concatenate([ci, di, jnp.zeros((6,), jnp.int32),
                             neighbor_indices.astype(jnp.int32)])

    wct = head_W1[:, 0:H].T
    wdt = head_W1[:, H:2 * H].T
    wnt = head_W1[:, 2 * H:3 * H].T

    q = _head(idx40, h1, a20, a21,
              mlp2_W1.T, mlp2_b1[None, :], mlp2_W2.T, mlp2_b2[None, :],
              wct, wdt, wnt, head_b1[None, :], head_W2, head_b2[None, :])
    return q[:, 0]


# speedup vs baseline: 1.3957x; 1.0014x over previous
import functools

import jax
import jax.numpy as jnp
from jax import lax
from jax.experimental import pallas as pl
from jax.experimental.pallas import tpu as pltpu
from jax.experimental.pallas import tpu_sc as plsc

N = 10000
E = 320000
IN = 128
H = 128
ED = 16
K = 32

NC = 2
NS = 16
NW = NC * NS
C = 64
CPW = 159
EPW = C * CPW
E_PAD = NW * EPW
N_PAD = 10112
ROWS_PER_TILE = N_PAD // NS
REP = 8


def _sc_msg_pass(h_hbm, e_hbm, src_hbm, dst_hbm, out_hbm,
                 sidx0, sidx1, sidx2, didx0, didx1, didx2,
                 ebuf0, ebuf1, ebuf2, hbuf0, hbuf1, hbuf2,
                 aggr,
                 se0, se1, se2, sh0, sh1, sh2,
                 si0, si1, si2):
    cid = lax.axis_index("c")
    sid = lax.axis_index("s")
    wid = sid * NC + cid
    base0 = wid * EPW

    sidx = [sidx0, sidx1, sidx2]
    didx = [didx0, didx1, didx2]
    ebuf = [ebuf0, ebuf1, ebuf2]
    hbuf = [hbuf0, hbuf1, hbuf2]
    sem_e = [se0, se1, se2]
    sem_h = [sh0, sh1, sh2]
    sem_i = [si0, si1, si2]

    zeros16 = jnp.zeros((16,), jnp.float32)

    @plsc.parallel_loop(0, C, unroll=4)
    def _(j):
        for g in range(H // 16):
            ebuf0[j, pl.ds(g * 16, 16)] = zeros16

    row0 = sid * ROWS_PER_TILE
    full = ROWS_PER_TILE // C
    rem = ROWS_PER_TILE - full * C
    for r in range(full):
        pltpu.sync_copy(ebuf0, aggr.at[pl.ds(row0 + r * C, C)])
    pltpu.sync_copy(ebuf0.at[pl.ds(0, rem)], aggr.at[pl.ds(row0 + full * C, rem)])
    plsc.subcore_barrier()

    def start_idx(chunk, b):
        pltpu.async_copy(src_hbm.at[pl.ds(base0 + chunk * C, C)], sidx[b], sem_i[b])
        pltpu.async_copy(dst_hbm.at[pl.ds(base0 + chunk * C, C)], didx[b], sem_i[b])

    def wait_idx(b):
        pltpu.make_async_copy(src_hbm.at[pl.ds(0, C)], sidx[b], sem_i[b]).wait()
        pltpu.make_async_copy(dst_hbm.at[pl.ds(0, C)], didx[b], sem_i[b]).wait()

    def start_streams(chunk, b):
        pltpu.async_copy(e_hbm.at[pl.ds(base0 + chunk * C, C)], ebuf[b], sem_e[b])
        pltpu.async_copy(h_hbm.at[sidx[b]], hbuf[b], sem_h[b])

    def wait_streams(b):
        pltpu.make_async_copy(e_hbm.at[pl.ds(0, C)], ebuf[b], sem_e[b]).wait()
        pltpu.make_async_copy(h_hbm.at[sidx[b]], hbuf[b], sem_h[b]).wait()

    start_idx(0, 0)
    wait_idx(0)
    start_streams(0, 0)

    NG = CPW // 3

    def pipe_body(g, _):
        for k in range(3):
            i = 3 * g + k
            cur = k
            nxt = (k + 1) % 3

            def launch_streams(i=i, nxt=nxt):
                wait_idx(nxt)
                start_streams(i + 1, nxt)

            last = k == 2
            if not last:
                start_idx(i + 1, nxt)
            else:
                @pl.when(g < NG - 1)
                def _():
                    start_idx(i + 1, nxt)

            wait_streams(cur)

            @plsc.parallel_loop(0, C, unroll=4)
            def _(j, cur=cur):
                for gg in range(H // 16):
                    sl = pl.ds(gg * 16, 16)
                    ebuf[cur][j, sl] = jnp.maximum(
                        ebuf[cur][j, sl] + hbuf[cur][j, sl], 0.0)

            if not last:
                launch_streams()
            else:
                @pl.when(g < NG - 1)
                def _():
                    launch_streams()

            pltpu.sync_copy(ebuf[cur], aggr.at[didx[cur]], add=True)
        return 0

    lax.fori_loop(0, NG, pipe_body, 0)
    plsc.subcore_barrier()

    out_row0 = cid * N_PAD + row0
    for r in range(full):
        pltpu.sync_copy(aggr.at[pl.ds(row0 + r * C, C)], hbuf0)
        pltpu.sync_copy(hbuf0, out_hbm.at[pl.ds(out_row0 + r * C, C)])
    pltpu.sync_copy(aggr.at[pl.ds(row0 + full * C, rem)], hbuf0.at[pl.ds(0, rem)])
    pltpu.sync_copy(hbuf0.at[pl.ds(0, rem)], out_hbm.at[pl.ds(out_row0 + full * C, rem)])


_sc_mesh = plsc.VectorSubcoreMesh(core_axis_name="c", subcore_axis_name="s",
                                  num_cores=NC, num_subcores=NS)

_sc_msg_pass_call = functools.partial(
    pl.kernel,
    out_type=jax.ShapeDtypeStruct((NC * N_PAD, H), jnp.float32),
    mesh=_sc_mesh,
    scratch_types=(
        [pltpu.VMEM((C,), jnp.int32)] * 6
        + [pltpu.VMEM((C, H), jnp.float32)] * 6
        + [pltpu.VMEM_SHARED((N_PAD, H), jnp.float32)]
        + [pltpu.SemaphoreType.DMA] * 9
    ),
)(_sc_msg_pass)


def _edgelin_body(ea_ref, w1t_ref, b1_ref, e1_ref):
    a = ea_ref[...]
    e1_ref[...] = jnp.dot(a, w1t_ref[...], preferred_element_type=jnp.float32) + b1_ref[...]


_BE = 1024


def _edgelin(ea_pad, w1t, b1):
    grid = (E_PAD // _BE,)
    return pl.pallas_call(
        _edgelin_body,
        grid=grid,
        in_specs=[
            pl.BlockSpec((_BE, ED), lambda i: (i, 0)),
            pl.BlockSpec((ED, H), lambda i: (0, 0)),
            pl.BlockSpec((1, H), lambda i: (0, 0)),
        ],
        out_specs=pl.BlockSpec((_BE, H), lambda i: (i, 0)),
        out_shape=jax.ShapeDtypeStruct((E_PAD, H), jnp.float32),
    )(ea_pad, w1t, b1)


def _replicate_body(x_ref, out_ref):
    out_ref[...] = x_ref[...]


def _replicate(x):
    return pl.pallas_call(
        _replicate_body,
        grid=(REP,),
        in_specs=[pl.BlockSpec((N, H), lambda r: (0, 0))],
        out_specs=pl.BlockSpec((N, H), lambda r: (r, 0)),
        out_shape=jax.ShapeDtypeStruct((REP * N, H), jnp.float32),
    )(x)


def _mlp1_body(x_ref, a0_ref, a1_ref, w1t_ref, b1_ref, w2t_ref, b2_ref, out_ref):
    z = x_ref[...] + a0_ref[...] + a1_ref[...]
    h = jax.nn.relu(jnp.dot(z, w1t_ref[...], preferred_element_type=jnp.float32) + b1_ref[...])
    o = jnp.dot(h, w2t_ref[...], preferred_element_type=jnp.float32) + b2_ref[...]
    out_ref[...] = jax.nn.relu(o)


def _mlp1(x, a0, a1, w1t, b1, w2t, b2):
    return pl.pallas_call(
        _mlp1_body,
        out_shape=jax.ShapeDtypeStruct((N, H), jnp.float32),
    )(x, a0, a1, w1t, b1, w2t, b2)


def _head_body(idx_ref, h1_ref, a0_ref, a1_ref,
               mw1t_ref, mb1_ref, mw2t_ref, mb2_ref,
               wct_ref, wdt_ref, wnt_ref, hb1_ref, hw2_ref, hb2_ref,
               out_ref, rows_ref):
    for k in [0, 1] + list(range(8, 40)):
        idx = idx_ref[k]
        r = (h1_ref[pl.ds(idx, 1), :] + a0_ref[pl.ds(idx, 1), :]
             + a1_ref[pl.ds(idx, 1), :])
        rows_ref[pl.ds(k, 1), :] = r
    rows = rows_ref[...]
    hmid = jax.nn.relu(jnp.dot(rows, mw1t_ref[...], preferred_element_type=jnp.float32) + mb1_ref[...])
    h2r = jnp.dot(hmid, mw2t_ref[...], preferred_element_type=jnp.float32) + mb2_ref[...]
    curr = h2r[0:1, :]
    dest = h2r[1:2, :]
    nbr = h2r[8:40, :]
    base = (jnp.dot(curr, wct_ref[...], preferred_element_type=jnp.float32)
            + jnp.dot(dest, wdt_ref[...], preferred_element_type=jnp.float32)
            + hb1_ref[...])
    hh = jax.nn.relu(jnp.dot(nbr, wnt_ref[...], preferred_element_type=jnp.float32) + base)
    q = jnp.sum(hh * hw2_ref[...], axis=1, keepdims=True) + hb2_ref[0, 0]
    out_ref[...] = q


def _head(idx40, h1, a0, a1, mw1t, mb1, mw2t, mb2,
          wct, wdt, wnt, hb1, hw2, hb2):
    return pl.pallas_call(
        _head_body,
        in_specs=[
            pl.BlockSpec(memory_space=pltpu.SMEM),
            pl.BlockSpec((N, H), lambda: (0, 0)),
            pl.BlockSpec((N, H), lambda: (0, 0)),
            pl.BlockSpec((N, H), lambda: (0, 0)),
            pl.BlockSpec((H, H), lambda: (0, 0)),
            pl.BlockSpec((1, H), lambda: (0, 0)),
            pl.BlockSpec((H, H), lambda: (0, 0)),
            pl.BlockSpec((1, H), lambda: (0, 0)),
            pl.BlockSpec((H, H), lambda: (0, 0)),
            pl.BlockSpec((H, H), lambda: (0, 0)),
            pl.BlockSpec((H, H), lambda: (0, 0)),
            pl.BlockSpec((1, H), lambda: (0, 0)),
            pl.BlockSpec((1, H), lambda: (0, 0)),
            pl.BlockSpec((1, 1), lambda: (0, 0)),
        ],
        out_shape=jax.ShapeDtypeStruct((K, 1), jnp.float32),
        scratch_shapes=[pltpu.VMEM((40, H), jnp.float32)],
    )(idx40, h1, a0, a1, mw1t, mb1, mw2t, mb2, wct, wdt, wnt, hb1, hw2, hb2)


def kernel(x, edge_index, curr_idx, dest_idx, neighbor_indices, edge_attr,
           lin_e1_W, lin_e1_b, mlp1_W1, mlp1_b1, mlp1_W2, mlp1_b2,
           lin_e2_W, lin_e2_b, mlp2_W1, mlp2_b1, mlp2_W2, mlp2_b2,
           head_W1, head_b1, head_W2, head_b2):
    src = edge_index[0]
    dst = edge_index[1]
    pad = E_PAD - E
    src_pad = jnp.concatenate([src, jnp.zeros((pad,), jnp.int32)])
    dst_pad = jnp.concatenate([dst, jnp.full((pad,), N, jnp.int32)])
    ea_pad = jnp.concatenate([edge_attr, jnp.zeros((pad, ED), jnp.float32)])
    src_rep = src_pad + (jnp.arange(E_PAD, dtype=jnp.int32) % REP) * N

    e1 = _edgelin(ea_pad, lin_e1_W.T, lin_e1_b[None, :])

    x_rep = _replicate(x)
    a1_parts = _sc_msg_pass_call(x_rep, e1, src_rep, dst_pad)
    e2 = _edgelin(ea_pad, lin_e2_W.T, lin_e2_b[None, :])
    a10 = a1_parts[:N]
    a11 = a1_parts[N_PAD:N_PAD + N]

    h1 = _mlp1(x, a10, a11, mlp1_W1.T, mlp1_b1[None, :],
               mlp1_W2.T, mlp1_b2[None, :])

    h1_rep = _replicate(h1)
    a2_parts = _sc_msg_pass_call(h1_rep, e2, src_rep, dst_pad)
    a20 = a2_parts[:N]
    a21 = a2_parts[N_PAD:N_PAD + N]

    ci = jnp.asarray(curr_idx, jnp.int32)[None]
    di = jnp.asarray(dest_idx, jnp.int32)[None]
    idx40 = jnp.concatenate([ci, di, jnp.zeros((6,), jnp.int32),
                             neighbor_indices.astype(jnp.int32)])

    wct = head_W1[:, 0:H].T
    wdt = head_W1[:, H:2 * H].T
    wnt = head_W1[:, 2 * H:3 * H].T

    q = _head(idx40, h1, a20, a21,
              mlp2_W1.T, mlp2_b1[None, :], mlp2_W2.T, mlp2_b2[None, :],
              wct, wdt, wnt, head_b1[None, :], head_W2, head_b2[None, :])
    return q[:, 0]


# speedup vs baseline: 1.4587x; 1.0452x over previous
import functools

import jax
import jax.numpy as jnp
from jax import lax
from jax.experimental import pallas as pl
from jax.experimental.pallas import tpu as pltpu
from jax.experimental.pallas import tpu_sc as plsc

N = 10000
E = 320000
IN = 128
H = 128
ED = 16
K = 32

NC = 2
NS = 16
NW = NC * NS
C = 96
CPW = 106
EPW = C * CPW
E_PAD = NW * EPW
N_PAD = 10112
ROWS_PER_TILE = N_PAD // NS
REP = 8


def _sc_msg_pass(h_hbm, e_hbm, src_hbm, dst_hbm, out_hbm,
                 sidx0, sidx1, didx0, didx1,
                 ebuf0, ebuf1, hbuf0, hbuf1,
                 aggr,
                 se0, se1, sh0, sh1, si0, si1):
    cid = lax.axis_index("c")
    sid = lax.axis_index("s")
    wid = sid * NC + cid
    base0 = wid * EPW

    sidx = [sidx0, sidx1]
    didx = [didx0, didx1]
    ebuf = [ebuf0, ebuf1]
    hbuf = [hbuf0, hbuf1]
    sem_e = [se0, se1]
    sem_h = [sh0, sh1]
    sem_i = [si0, si1]

    zeros16 = jnp.zeros((16,), jnp.float32)

    @plsc.parallel_loop(0, C, unroll=4)
    def _(j):
        for g in range(H // 16):
            ebuf0[j, pl.ds(g * 16, 16)] = zeros16

    row0 = sid * ROWS_PER_TILE
    full = ROWS_PER_TILE // C
    rem = ROWS_PER_TILE - full * C
    for r in range(full):
        pltpu.sync_copy(ebuf0, aggr.at[pl.ds(row0 + r * C, C)])
    pltpu.sync_copy(ebuf0.at[pl.ds(0, rem)], aggr.at[pl.ds(row0 + full * C, rem)])
    plsc.subcore_barrier()

    def start_idx(chunk, b):
        pltpu.async_copy(src_hbm.at[pl.ds(base0 + chunk * C, C)], sidx[b], sem_i[b])
        pltpu.async_copy(dst_hbm.at[pl.ds(base0 + chunk * C, C)], didx[b], sem_i[b])

    def wait_idx(b):
        pltpu.make_async_copy(src_hbm.at[pl.ds(0, C)], sidx[b], sem_i[b]).wait()
        pltpu.make_async_copy(dst_hbm.at[pl.ds(0, C)], didx[b], sem_i[b]).wait()

    def start_streams(chunk, b):
        pltpu.async_copy(e_hbm.at[pl.ds(base0 + chunk * C, C)], ebuf[b], sem_e[b])
        pltpu.async_copy(h_hbm.at[sidx[b]], hbuf[b], sem_h[b])

    def wait_streams(b):
        pltpu.make_async_copy(e_hbm.at[pl.ds(0, C)], ebuf[b], sem_e[b]).wait()
        pltpu.make_async_copy(h_hbm.at[sidx[b]], hbuf[b], sem_h[b]).wait()

    start_idx(0, 0)
    wait_idx(0)
    start_streams(0, 0)

    NG = CPW // 2

    def pipe_body(g, _):
        for k in range(2):
            i = 2 * g + k
            cur = k
            nxt = 1 - k

            def launch_streams(i=i, nxt=nxt):
                wait_idx(nxt)
                start_streams(i + 1, nxt)

            last = k == 1
            if not last:
                start_idx(i + 1, nxt)
            else:
                @pl.when(g < NG - 1)
                def _():
                    start_idx(i + 1, nxt)

            wait_streams(cur)

            @plsc.parallel_loop(0, C, unroll=4)
            def _(j, cur=cur):
                for gg in range(H // 16):
                    sl = pl.ds(gg * 16, 16)
                    ebuf[cur][j, sl] = jnp.maximum(
                        ebuf[cur][j, sl] + hbuf[cur][j, sl], 0.0)

            if not last:
                launch_streams()
            else:
                @pl.when(g < NG - 1)
                def _():
                    launch_streams()

            pltpu.sync_copy(ebuf[cur], aggr.at[didx[cur]], add=True)
        return 0

    lax.fori_loop(0, NG, pipe_body, 0)
    plsc.subcore_barrier()

    out_row0 = cid * N_PAD + row0
    for r in range(full):
        pltpu.sync_copy(aggr.at[pl.ds(row0 + r * C, C)], hbuf0)
        pltpu.sync_copy(hbuf0, out_hbm.at[pl.ds(out_row0 + r * C, C)])
    pltpu.sync_copy(aggr.at[pl.ds(row0 + full * C, rem)], hbuf0.at[pl.ds(0, rem)])
    pltpu.sync_copy(hbuf0.at[pl.ds(0, rem)], out_hbm.at[pl.ds(out_row0 + full * C, rem)])


_sc_mesh = plsc.VectorSubcoreMesh(core_axis_name="c", subcore_axis_name="s",
                                  num_cores=NC, num_subcores=NS)

_sc_msg_pass_call = functools.partial(
    pl.kernel,
    out_type=jax.ShapeDtypeStruct((NC * N_PAD, H), jnp.float32),
    mesh=_sc_mesh,
    scratch_types=(
        [pltpu.VMEM((C,), jnp.int32)] * 4
        + [pltpu.VMEM((C, H), jnp.float32)] * 4
        + [pltpu.VMEM_SHARED((N_PAD, H), jnp.float32)]
        + [pltpu.SemaphoreType.DMA] * 6
    ),
)(_sc_msg_pass)


def _edgelin_body(ea_ref, w1t_ref, b1_ref, e1_ref):
    a = ea_ref[...]
    e1_ref[...] = jnp.dot(a, w1t_ref[...], preferred_element_type=jnp.float32) + b1_ref[...]


_BE = 1024


def _edgelin(ea_pad, w1t, b1):
    grid = (E_PAD // _BE,)
    return pl.pallas_call(
        _edgelin_body,
        grid=grid,
        in_specs=[
            pl.BlockSpec((_BE, ED), lambda i: (i, 0)),
            pl.BlockSpec((ED, H), lambda i: (0, 0)),
            pl.BlockSpec((1, H), lambda i: (0, 0)),
        ],
        out_specs=pl.BlockSpec((_BE, H), lambda i: (i, 0)),
        out_shape=jax.ShapeDtypeStruct((E_PAD, H), jnp.float32),
    )(ea_pad, w1t, b1)


def _replicate_body(x_ref, out_ref):
    out_ref[...] = x_ref[...]


def _replicate(x):
    return pl.pallas_call(
        _replicate_body,
        grid=(REP,),
        in_specs=[pl.BlockSpec((N, H), lambda r: (0, 0))],
        out_specs=pl.BlockSpec((N, H), lambda r: (r, 0)),
        out_shape=jax.ShapeDtypeStruct((REP * N, H), jnp.float32),
    )(x)


def _mlp1_body(x_ref, a0_ref, a1_ref, w1t_ref, b1_ref, w2t_ref, b2_ref, out_ref):
    z = x_ref[...] + a0_ref[...] + a1_ref[...]
    h = jax.nn.relu(jnp.dot(z, w1t_ref[...], preferred_element_type=jnp.float32) + b1_ref[...])
    o = jnp.dot(h, w2t_ref[...], preferred_element_type=jnp.float32) + b2_ref[...]
    out_ref[...] = jax.nn.relu(o)


def _mlp1(x, a0, a1, w1t, b1, w2t, b2):
    return pl.pallas_call(
        _mlp1_body,
        out_shape=jax.ShapeDtypeStruct((N, H), jnp.float32),
    )(x, a0, a1, w1t, b1, w2t, b2)


def _head_body(idx_ref, h1_ref, a0_ref, a1_ref,
               mw1t_ref, mb1_ref, mw2t_ref, mb2_ref,
               wct_ref, wdt_ref, wnt_ref, hb1_ref, hw2_ref, hb2_ref,
               out_ref, rows_ref):
    for k in [0, 1] + list(range(8, 40)):
        idx = idx_ref[k]
        r = (h1_ref[pl.ds(idx, 1), :] + a0_ref[pl.ds(idx, 1), :]
             + a1_ref[pl.ds(idx, 1), :])
        rows_ref[pl.ds(k, 1), :] = r
    rows = rows_ref[...]
    hmid = jax.nn.relu(jnp.dot(rows, mw1t_ref[...], preferred_element_type=jnp.float32) + mb1_ref[...])
    h2r = jnp.dot(hmid, mw2t_ref[...], preferred_element_type=jnp.float32) + mb2_ref[...]
    curr = h2r[0:1, :]
    dest = h2r[1:2, :]
    nbr = h2r[8:40, :]
    base = (jnp.dot(curr, wct_ref[...], preferred_element_type=jnp.float32)
            + jnp.dot(dest, wdt_ref[...], preferred_element_type=jnp.float32)
            + hb1_ref[...])
    hh = jax.nn.relu(jnp.dot(nbr, wnt_ref[...], preferred_element_type=jnp.float32) + base)
    q = jnp.sum(hh * hw2_ref[...], axis=1, keepdims=True) + hb2_ref[0, 0]
    out_ref[...] = q


def _head(idx40, h1, a0, a1, mw1t, mb1, mw2t, mb2,
          wct, wdt, wnt, hb1, hw2, hb2):
    return pl.pallas_call(
        _head_body,
        in_specs=[
            pl.BlockSpec(memory_space=pltpu.SMEM),
            pl.BlockSpec((N, H), lambda: (0, 0)),
            pl.BlockSpec((N, H), lambda: (0, 0)),
            pl.BlockSpec((N, H), lambda: (0, 0)),
            pl.BlockSpec((H, H), lambda: (0, 0)),
            pl.BlockSpec((1, H), lambda: (0, 0)),
            pl.BlockSpec((H, H), lambda: (0, 0)),
            pl.BlockSpec((1, H), lambda: (0, 0)),
            pl.BlockSpec((H, H), lambda: (0, 0)),
            pl.BlockSpec((H, H), lambda: (0, 0)),
            pl.BlockSpec((H, H), lambda: (0, 0)),
            pl.BlockSpec((1, H), lambda: (0, 0)),
            pl.BlockSpec((1, H), lambda: (0, 0)),
            pl.BlockSpec((1, 1), lambda: (0, 0)),
        ],
        out_shape=jax.ShapeDtypeStruct((K, 1), jnp.float32),
        scratch_shapes=[pltpu.VMEM((40, H), jnp.float32)],
    )(idx40, h1, a0, a1, mw1t, mb1, mw2t, mb2, wct, wdt, wnt, hb1, hw2, hb2)


def kernel(x, edge_index, curr_idx, dest_idx, neighbor_indices, edge_attr,
           lin_e1_W, lin_e1_b, mlp1_W1, mlp1_b1, mlp1_W2, mlp1_b2,
           lin_e2_W, lin_e2_b, mlp2_W1, mlp2_b1, mlp2_W2, mlp2_b2,
           head_W1, head_b1, head_W2, head_b2):
    src = edge_index[0]
    dst = edge_index[1]
    pad = E_PAD - E
    src_pad = jnp.concatenate([src, jnp.zeros((pad,), jnp.int32)])
    dst_pad = jnp.concatenate([dst, jnp.full((pad,), N, jnp.int32)])
    ea_pad = jnp.concatenate([edge_attr, jnp.zeros((pad, ED), jnp.float32)])
    src_rep = src_pad + (jnp.arange(E_PAD, dtype=jnp.int32) % REP) * N

    e1 = _edgelin(ea_pad, lin_e1_W.T, lin_e1_b[None, :])

    x_rep = _replicate(x)
    a1_parts = _sc_msg_pass_call(x_rep, e1, src_rep, dst_pad)
    e2 = _edgelin(ea_pad, lin_e2_W.T, lin_e2_b[None, :])
    a10 = a1_parts[:N]
    a11 = a1_parts[N_PAD:N_PAD + N]

    h1 = _mlp1(x, a10, a11, mlp1_W1.T, mlp1_b1[None, :],
               mlp1_W2.T, mlp1_b2[None, :])

    h1_rep = _replicate(h1)
    a2_parts = _sc_msg_pass_call(h1_rep, e2, src_rep, dst_pad)
    a20 = a2_parts[:N]
    a21 = a2_parts[N_PAD:N_PAD + N]

    ci = jnp.asarray(curr_idx, jnp.int32)[None]
    di = jnp.asarray(dest_idx, jnp.int32)[None]
    idx40 = jnp.concatenate([ci, di, jnp.zeros((6,), jnp.int32),
                             neighbor_indices.astype(jnp.int32)])

    wct = head_W1[:, 0:H].T
    wdt = head_W1[:, H:2 * H].T
    wnt = head_W1[:, 2 * H:3 * H].T

    q = _head(idx40, h1, a20, a21,
              mlp2_W1.T, mlp2_b1[None, :], mlp2_W2.T, mlp2_b2[None, :],
              wct, wdt, wnt, head_b1[None, :], head_W2, head_b2[None, :])
    return q[:, 0]
